# baseline (speedup 1.0000x reference)
import functools

import jax
import jax.numpy as jnp
from jax.experimental import pallas as pl
from jax.experimental.pallas import tpu as pltpu

_N = 100000
_D = 128
_K = 32
_B = 10000
_NB = _N // _B


def _body(x_ref, evt_ref, out_ref):
    out_ref[...] = x_ref[...] * 2.0 + evt_ref[0, 0, 0]


@functools.partial(jax.jit, static_argnames=())
def kernel(x, eigvec, eigval, W_filter, b_filter, W_out, b_out):
    evt = jnp.transpose(eigvec.reshape(_NB, _B, _K), (0, 2, 1))
    out = pl.pallas_call(
        _body,
        grid=(_NB,),
        in_specs=[
            pl.BlockSpec((_B, _D), lambda i: (i, 0)),
            pl.BlockSpec((1, _K, _B), lambda i: (i, 0, 0)),
        ],
        out_specs=pl.BlockSpec((_B, _D), lambda i: (i, 0)),
        out_shape=jax.ShapeDtypeStruct((_N, _D), jnp.float32),
        compiler_params=pltpu.CompilerParams(
            dimension_semantics=("arbitrary",),
        ),
    )(x, evt)
    return out


# speedup vs baseline: 1.7397x; 1.7397x over previous
import functools

import jax
import jax.numpy as jnp
from jax.experimental import pallas as pl
from jax.experimental.pallas import tpu as pltpu

_N = 100000
_D = 128
_K = 32
_B = 4096
_NB = (_N + _B - 1) // _B


def _body(evt_ref, x_ref, eigval_ref, wf_ref, bf_ref, wo_ref, bo_ref,
          out_ref, acc_ref, spec2_ref, x_cache):
    p = pl.program_id(0)
    i = pl.program_id(1)

    @pl.when(jnp.logical_and(p == 0, i == 0))
    def _init():
        acc_ref[...] = jnp.zeros_like(acc_ref)

    @pl.when(jnp.logical_and(p == 0, i < _NB - 1))
    def _accumulate():
        ev16 = evt_ref[:, pl.ds(i * _B, _B)].astype(jnp.bfloat16)
        x16 = x_ref[...].astype(jnp.bfloat16)
        acc_ref[...] += jnp.dot(ev16, x16, preferred_element_type=jnp.float32)
        x_cache[pl.ds(i * _B, _B), :] = x16

    @pl.when(jnp.logical_and(p == 0, i == _NB - 1))
    def _accumulate_tail():
        nvalid = _N - (_NB - 1) * _B
        ev = evt_ref[:, pl.ds(i * _B, _B)]
        lane = jax.lax.broadcasted_iota(jnp.int32, (_K, _B), 1)
        ev16 = jnp.where(lane < nvalid, ev, 0.0).astype(jnp.bfloat16)
        row = jax.lax.broadcasted_iota(jnp.int32, (_B, _D), 0)
        x16 = jnp.where(row < nvalid, x_ref[...], 0.0).astype(jnp.bfloat16)
        acc_ref[...] += jnp.dot(ev16, x16, preferred_element_type=jnp.float32)
        x_cache[pl.ds(i * _B, _B), :] = x16

    @pl.when(jnp.logical_and(p == 1, i == 0))
    def _spectral():
        spec = acc_ref[...] * eigval_ref[...]
        gate = jax.nn.sigmoid(
            jnp.dot(spec, wf_ref[...], preferred_element_type=jnp.float32)
            + bf_ref[...]
        )
        spec = spec * gate
        spec2_ref[...] = jnp.dot(
            spec, wo_ref[...], preferred_element_type=jnp.float32
        ).astype(jnp.bfloat16)

    @pl.when(p == 1)
    def _backproject():
        ev16 = evt_ref[:, pl.ds(i * _B, _B)].astype(jnp.bfloat16)
        proj = jax.lax.dot_general(
            ev16, spec2_ref[...],
            dimension_numbers=(((0,), (0,)), ((), ())),
            preferred_element_type=jnp.float32,
        )
        out_ref[...] = (
            x_cache[pl.ds(i * _B, _B), :].astype(jnp.float32)
            + proj
            + bo_ref[...]
        )


@functools.partial(jax.jit, static_argnames=())
def kernel(x, eigvec, eigval, W_filter, b_filter, W_out, b_out):
    evt = eigvec.T
    eigval2 = eigval.reshape(_K, 1)
    bf2 = b_filter.reshape(1, _D)
    bo2 = b_out.reshape(1, _D)

    out = pl.pallas_call(
        _body,
        grid=(2, _NB),
        in_specs=[
            pl.BlockSpec((_K, _NB * _B), lambda p, i: (0, 0)),

            pl.BlockSpec((_B, _D),
                         lambda p, i: (jnp.where(p == 0, i, _NB - 1), 0)),
            pl.BlockSpec((_K, 1), lambda p, i: (0, 0)),
            pl.BlockSpec((_D, _D), lambda p, i: (0, 0)),
            pl.BlockSpec((1, _D), lambda p, i: (0, 0)),
            pl.BlockSpec((_D, _D), lambda p, i: (0, 0)),
            pl.BlockSpec((1, _D), lambda p, i: (0, 0)),
        ],
        out_specs=pl.BlockSpec((_B, _D),
                               lambda p, i: (jnp.where(p == 0, 0, i), 0)),
        out_shape=jax.ShapeDtypeStruct((_N, _D), jnp.float32),
        scratch_shapes=[
            pltpu.VMEM((_K, _D), jnp.float32),
            pltpu.VMEM((_K, _D), jnp.bfloat16),
            pltpu.VMEM((_NB * _B, _D), jnp.bfloat16),
        ],
        compiler_params=pltpu.CompilerParams(
            dimension_semantics=("arbitrary", "arbitrary"),
            vmem_limit_bytes=64 * 1024 * 1024,
        ),
    )(evt, x, eigval2, W_filter, bf2, W_out, bo2)
    return out


# speedup vs baseline: 2.1880x; 1.2577x over previous
import functools

import jax
import jax.numpy as jnp
from jax.experimental import pallas as pl
from jax.experimental.pallas import tpu as pltpu

_N = 100000
_D = 128
_K = 32
_B = 8192
_NB = (_N + _B - 1) // _B


def _body(evt_ref, x_ref, eigval_ref, wf_ref, bf_ref, wo_ref, bo_ref,
          out_ref, acc_ref, spec2_ref, x_cache):
    p = pl.program_id(0)
    i = pl.program_id(1)

    @pl.when(jnp.logical_and(p == 0, i == 0))
    def _init():
        acc_ref[...] = jnp.zeros_like(acc_ref)

    @pl.when(jnp.logical_and(p == 0, i < _NB - 1))
    def _accumulate():
        ev16 = evt_ref[:, pl.ds(i * _B, _B)].astype(jnp.bfloat16)
        x16 = x_ref[...].astype(jnp.bfloat16)
        acc_ref[...] += jnp.dot(ev16, x16, preferred_element_type=jnp.float32)
        x_cache[pl.ds(i * _B, _B), :] = x16

    @pl.when(jnp.logical_and(p == 0, i == _NB - 1))
    def _accumulate_tail():
        nvalid = _N - (_NB - 1) * _B
        ev = evt_ref[:, pl.ds(i * _B, _B)]
        lane = jax.lax.broadcasted_iota(jnp.int32, (_K, _B), 1)
        ev16 = jnp.where(lane < nvalid, ev, 0.0).astype(jnp.bfloat16)
        row = jax.lax.broadcasted_iota(jnp.int32, (_B, _D), 0)
        x16 = jnp.where(row < nvalid, x_ref[...], 0.0).astype(jnp.bfloat16)
        acc_ref[...] += jnp.dot(ev16, x16, preferred_element_type=jnp.float32)
        x_cache[pl.ds(i * _B, _B), :] = x16

    @pl.when(jnp.logical_and(p == 1, i == 0))
    def _spectral():
        spec = acc_ref[...] * eigval_ref[...]
        gate = jax.nn.sigmoid(
            jnp.dot(spec, wf_ref[...], preferred_element_type=jnp.float32)
            + bf_ref[...]
        )
        spec = spec * gate
        spec2_ref[...] = jnp.dot(
            spec, wo_ref[...], preferred_element_type=jnp.float32
        ).astype(jnp.bfloat16)

    @pl.when(p == 1)
    def _backproject():
        ev16 = evt_ref[:, pl.ds(i * _B, _B)].astype(jnp.bfloat16)
        proj = jax.lax.dot_general(
            ev16, spec2_ref[...],
            dimension_numbers=(((0,), (0,)), ((), ())),
            preferred_element_type=jnp.float32,
        )
        out_ref[...] = (
            x_cache[pl.ds(i * _B, _B), :].astype(jnp.float32)
            + proj
            + bo_ref[...]
        )


@functools.partial(jax.jit, static_argnames=())
def kernel(x, eigvec, eigval, W_filter, b_filter, W_out, b_out):
    evt = eigvec.T
    eigval2 = eigval.reshape(_K, 1)
    bf2 = b_filter.reshape(1, _D)
    bo2 = b_out.reshape(1, _D)

    out = pl.pallas_call(
        _body,
        grid=(2, _NB),
        in_specs=[
            pl.BlockSpec((_K, _NB * _B), lambda p, i: (0, 0)),

            pl.BlockSpec((_B, _D),
                         lambda p, i: (jnp.where(p == 0, i, _NB - 1), 0)),
            pl.BlockSpec((_K, 1), lambda p, i: (0, 0)),
            pl.BlockSpec((_D, _D), lambda p, i: (0, 0)),
            pl.BlockSpec((1, _D), lambda p, i: (0, 0)),
            pl.BlockSpec((_D, _D), lambda p, i: (0, 0)),
            pl.BlockSpec((1, _D), lambda p, i: (0, 0)),
        ],
        out_specs=pl.BlockSpec((_B, _D),
                               lambda p, i: (jnp.where(p == 0, 0, i), 0)),
        out_shape=jax.ShapeDtypeStruct((_N, _D), jnp.float32),
        scratch_shapes=[
            pltpu.VMEM((_K, _D), jnp.float32),
            pltpu.VMEM((_K, _D), jnp.bfloat16),
            pltpu.VMEM((_NB * _B, _D), jnp.bfloat16),
        ],
        compiler_params=pltpu.CompilerParams(
            dimension_semantics=("arbitrary", "arbitrary"),
            vmem_limit_bytes=64 * 1024 * 1024,
        ),
    )(evt, x, eigval2, W_filter, bf2, W_out, bo2)
    return out
